# Initial kernel scaffold; baseline (speedup 1.0000x reference)
import jax
import jax.numpy as jnp
from jax.experimental import pallas as pl


def kernel(text, visual, audio, bank, W_t, b_t, W_v, b_v, W_a, b_a, prototypes):
    raise NotImplementedError("write your pallas kernel here")


# speedup vs baseline: 1.5831x; 1.5831x over previous
import functools

import jax
import jax.numpy as jnp
from jax import lax
from jax.experimental import pallas as pl
from jax.experimental.pallas import tpu as pltpu
from jax.experimental.pallas import tpu_sc as plsc

_K = 10
_TEMP = 0.1
_EPS_LN = 1e-5
_EPS_L2 = 1e-12


def _proj_kernel(t_ref, v_ref, a_ref, wt_ref, wv_ref, wa_ref,
                 bt_ref, bv_ref, ba_ref, yt_ref, yv_ref, ya_ref):
    def proj(x, w, b):
        return jnp.dot(x, w, preferred_element_type=jnp.float32) + b

    yt_ref[...] = proj(t_ref[...], wt_ref[...], bt_ref[...])
    yv_ref[...] = proj(v_ref[...], wv_ref[...], bv_ref[...])
    ya_ref[...] = proj(a_ref[...], wa_ref[...], ba_ref[...])


def _topk_kernel(qn_ref, bank_ref, vals_ref, idx_ref, *, n_tile, n_total):
    step = pl.program_id(0)

    @pl.when(step == 0)
    def _init():
        vals_ref[...] = jnp.full(vals_ref.shape, -jnp.inf, jnp.float32)
        idx_ref[...] = jnp.zeros(idx_ref.shape, jnp.int32)

    bn = bank_ref[...]
    s = lax.dot_general(qn_ref[...], bn, (((1,), (1,)), ((), ())),
                        preferred_element_type=jnp.float32)
    col0 = step * n_tile
    liota = lax.broadcasted_iota(jnp.int32, s.shape, 1)
    s = jnp.where(liota + col0 < n_total, s, -jnp.inf)

    tv, ti = [], []
    for _ in range(_K):
        m = jnp.max(s, axis=1, keepdims=True)
        a = jnp.min(jnp.where(s == m, liota, n_tile), axis=1, keepdims=True)
        tv.append(m)
        ti.append(a + col0)
        s = jnp.where(liota == a, -jnp.inf, s)
    cv = jnp.concatenate([vals_ref[...]] + tv, axis=1)
    ci = jnp.concatenate([idx_ref[...]] + ti, axis=1)

    miota = lax.broadcasted_iota(jnp.int32, cv.shape, 1)
    nv, ni = [], []
    for _ in range(_K):
        m = jnp.max(cv, axis=1, keepdims=True)
        a = jnp.min(jnp.where(cv == m, miota, 2 * _K), axis=1, keepdims=True)
        nv.append(m)
        ni.append(jnp.sum(jnp.where(miota == a, ci, 0), axis=1, keepdims=True))
        cv = jnp.where(miota == a, -jnp.inf, cv)
    vals_ref[...] = jnp.concatenate(nv, axis=1)
    idx_ref[...] = jnp.concatenate(ni, axis=1)


def _gather_sc(bank, idx_flat):
    info = plsc.get_sparse_core_info()
    nw = info.num_cores * info.num_subcores
    n, d = idx_flat.shape[0], bank.shape[1]
    b_per_w = n // nw
    ch = 64
    n_ch = b_per_w // ch
    mesh = plsc.VectorSubcoreMesh(core_axis_name="c", subcore_axis_name="s")

    @functools.partial(
        pl.kernel, mesh=mesh,
        out_type=jax.ShapeDtypeStruct((n, d), jnp.float32),
        scratch_types=[
            pltpu.VMEM((ch,), jnp.int32),
            pltpu.VMEM((ch, d), jnp.float32),
            pltpu.SemaphoreType.DMA,
        ],
    )
    def k(bank_hbm, idx_hbm, out_hbm, idx_v, rows_v, sem):
        wid = lax.axis_index("s") * info.num_cores + lax.axis_index("c")
        base = wid * b_per_w

        def body(c, carry):
            off = base + c * ch
            pltpu.sync_copy(idx_hbm.at[pl.ds(off, ch)], idx_v)
            pltpu.async_copy(bank_hbm.at[idx_v], rows_v, sem).wait()
            pltpu.sync_copy(rows_v, out_hbm.at[pl.ds(off, ch)])
            return carry

        lax.fori_loop(0, n_ch, body, 0)

    return k(bank, idx_flat)


def _agg_kernel(vals_ref, fused_ref, neigh_ref, proto_ref, logits_ref):
    w = jax.nn.softmax(vals_ref[...] / _TEMP, axis=-1)
    retrieved = jnp.sum(w[:, :, None] * neigh_ref[...], axis=1)
    out = fused_ref[...] + retrieved
    on = out / (jnp.sqrt(jnp.sum(out * out, axis=-1, keepdims=True)) + _EPS_L2)
    p = proto_ref[...]
    pn = p / (jnp.sqrt(jnp.sum(p * p, axis=-1, keepdims=True)) + _EPS_L2)
    logits_ref[...] = lax.dot_general(
        on, pn, (((1,), (1,)), ((), ())),
        preferred_element_type=jnp.float32) / _TEMP


def _run_proj(text, visual, audio, W_t, b_t, W_v, b_v, W_a, b_a):
    B = text.shape[0]
    D = W_t.shape[1]
    bt2 = b_t.reshape(1, D)
    bv2 = b_v.reshape(1, D)
    ba2 = b_a.reshape(1, D)

    b_tile = min(256, B)
    g1 = B // b_tile
    return pl.pallas_call(
        _proj_kernel,
        grid=(g1,),
        in_specs=[
            pl.BlockSpec((b_tile, text.shape[1]), lambda i: (i, 0)),
            pl.BlockSpec((b_tile, visual.shape[1]), lambda i: (i, 0)),
            pl.BlockSpec((b_tile, audio.shape[1]), lambda i: (i, 0)),
            pl.BlockSpec(W_t.shape, lambda i: (0, 0)),
            pl.BlockSpec(W_v.shape, lambda i: (0, 0)),
            pl.BlockSpec(W_a.shape, lambda i: (0, 0)),
            pl.BlockSpec((1, D), lambda i: (0, 0)),
            pl.BlockSpec((1, D), lambda i: (0, 0)),
            pl.BlockSpec((1, D), lambda i: (0, 0)),
        ],
        out_specs=[
            pl.BlockSpec((b_tile, D), lambda i: (i, 0)),
            pl.BlockSpec((b_tile, D), lambda i: (i, 0)),
            pl.BlockSpec((b_tile, D), lambda i: (i, 0)),
        ],
        out_shape=[
            jax.ShapeDtypeStruct((B, D), jnp.float32),
            jax.ShapeDtypeStruct((B, D), jnp.float32),
            jax.ShapeDtypeStruct((B, D), jnp.float32),
        ],
    )(text, visual, audio, W_t, W_v, W_a, bt2, bv2, ba2)


def _run_topk(qn, bn):
    B, D = qn.shape
    n_bank = bn.shape[0]
    n_tile = min(2048, n_bank)
    g2 = pl.cdiv(n_bank, n_tile)
    return pl.pallas_call(
        functools.partial(_topk_kernel, n_tile=n_tile, n_total=n_bank),
        grid=(g2,),
        in_specs=[
            pl.BlockSpec((B, D), lambda i: (0, 0)),
            pl.BlockSpec((n_tile, D), lambda i: (i, 0)),
        ],
        out_specs=[
            pl.BlockSpec((B, _K), lambda i: (0, 0)),
            pl.BlockSpec((B, _K), lambda i: (0, 0)),
        ],
        out_shape=[
            jax.ShapeDtypeStruct((B, _K), jnp.float32),
            jax.ShapeDtypeStruct((B, _K), jnp.int32),
        ],
    )(qn, bn)


def _run_agg(vals, fused, neigh, prototypes):
    B, D = fused.shape
    C = prototypes.shape[0]
    b_tile = min(256, B)
    g1 = B // b_tile
    return pl.pallas_call(
        _agg_kernel,
        grid=(g1,),
        in_specs=[
            pl.BlockSpec((b_tile, _K), lambda i: (i, 0)),
            pl.BlockSpec((b_tile, D), lambda i: (i, 0)),
            pl.BlockSpec((b_tile, _K, D), lambda i: (i, 0, 0)),
            pl.BlockSpec((C, D), lambda i: (0, 0)),
        ],
        out_specs=pl.BlockSpec((b_tile, C), lambda i: (i, 0)),
        out_shape=jax.ShapeDtypeStruct((B, C), jnp.float32),
    )(vals, fused, neigh, prototypes)


def _layernorm(x, eps=_EPS_LN):
    m = jnp.mean(x, axis=-1, keepdims=True)
    v = jnp.var(x, axis=-1, keepdims=True)
    return (x - m) / jnp.sqrt(v + eps)


def _l2norm(x, eps=_EPS_L2):
    return x / (jnp.linalg.norm(x, axis=-1, keepdims=True) + eps)


def kernel(text, visual, audio, bank, W_t, b_t, W_v, b_v, W_a, b_a, prototypes):
    B = text.shape[0]
    D = W_t.shape[1]
    yt, yv, ya = _run_proj(text, visual, audio, W_t, b_t, W_v, b_v, W_a, b_a)
    t = jax.nn.relu(_layernorm(yt))
    v = jax.nn.relu(_layernorm(yv))
    a = jax.nn.relu(_layernorm(ya))
    fused = (t + v + a) / 3.0
    qn = _l2norm(fused)
    bn = _l2norm(bank)
    vals, idx = _run_topk(qn, bn)
    neigh = _gather_sc(bank, idx.reshape(-1)).reshape(B, _K, D)
    logits = _run_agg(vals, fused, neigh, prototypes)
    return (logits, vals, idx)
